# baseline (speedup 1.0000x reference)
import jax
import jax.numpy as jnp
from jax.experimental import pallas as pl
from jax.experimental.pallas import tpu as pltpu

_B, _D, _N = 32, 64, 576
_K = 1024
_G = 72


def _vq_kernel(x_ref, cbm2_ref, e2_ref, gath_ref, out_ref, idx_ref):
    x = x_ref[0]
    s = jax.lax.dot_general(
        cbm2_ref[...], x, (((1,), (0,)), ((), ())),
        preferred_element_type=jnp.float32)
    d2 = s + e2_ref[...]
    m = jnp.min(d2, axis=0, keepdims=True)
    ohf = jnp.where(d2 <= m, 1.0, 0.0)
    r = jax.lax.dot_general(
        gath_ref[...], ohf, (((0,), (0,)), ((), ())),
        preferred_element_type=jnp.float32)
    out_ref[0] = r[0:_D, :]
    idx = r[_D, :] * 32.0 + r[_D + 1, :]
    idx_ref[0, 0, :] = idx.astype(jnp.int32)


def kernel(x_in, codebook):
    cb = codebook.astype(jnp.float32)
    cbm2 = -2.0 * cb
    e2 = jnp.sum(cb * cb, axis=1, keepdims=True)
    k_iota = jnp.arange(_K, dtype=jnp.float32)
    gath = jnp.concatenate(
        [cb,
         jnp.floor(k_iota / 32.0)[:, None],
         jnp.mod(k_iota, 32.0)[:, None],
         jnp.zeros((_K, _G - _D - 2), jnp.float32)], axis=1)
    out, idx = pl.pallas_call(
        _vq_kernel,
        grid=(_B,),
        in_specs=[
            pl.BlockSpec((1, _D, _N), lambda b: (b, 0, 0)),
            pl.BlockSpec((_K, _D), lambda b: (0, 0)),
            pl.BlockSpec((_K, 1), lambda b: (0, 0)),
            pl.BlockSpec((_K, _G), lambda b: (0, 0)),
        ],
        out_specs=[
            pl.BlockSpec((1, _D, _N), lambda b: (b, 0, 0)),
            pl.BlockSpec((1, 1, _N), lambda b: (b, 0, 0)),
        ],
        out_shape=[
            jax.ShapeDtypeStruct((_B, _D, _N), jnp.float32),
            jax.ShapeDtypeStruct((_B, 1, _N), jnp.int32),
        ],
        compiler_params=pltpu.CompilerParams(
            dimension_semantics=("parallel",)),
    )(x_in, cbm2, e2, gath)
    return out, jnp.reshape(idx, (_B, _N, 1))


# speedup vs baseline: 1.0579x; 1.0579x over previous
import jax
import jax.numpy as jnp
from jax.experimental import pallas as pl
from jax.experimental.pallas import tpu as pltpu

_B, _D, _N = 32, 64, 576
_K = 1024
_G = 72


def _vq_kernel(x_ref, cbm2_ref, e2_ref, emb_ref, out_ref, idx_ref):
    x = x_ref[0]
    s = jax.lax.dot_general(
        cbm2_ref[...], x, (((1,), (0,)), ((), ())),
        preferred_element_type=jnp.float32)
    d2 = s + e2_ref[...]
    idx = jnp.argmin(d2, axis=0)
    idx_ref[0, 0, :] = idx
    onehot = (jax.lax.broadcasted_iota(jnp.int32, (_K, _N), 0)
              == idx[None, :]).astype(jnp.float32)
    out = jax.lax.dot_general(
        emb_ref[...], onehot, (((0,), (0,)), ((), ())),
        preferred_element_type=jnp.float32)
    out_ref[0] = out


def kernel(x_in, codebook):
    cb = codebook.astype(jnp.float32)
    cbm2 = -2.0 * cb
    e2 = jnp.sum(cb * cb, axis=1, keepdims=True)
    out, idx = pl.pallas_call(
        _vq_kernel,
        grid=(_B,),
        in_specs=[
            pl.BlockSpec((1, _D, _N), lambda b: (b, 0, 0)),
            pl.BlockSpec((_K, _D), lambda b: (0, 0)),
            pl.BlockSpec((_K, 1), lambda b: (0, 0)),
            pl.BlockSpec((_K, _D), lambda b: (0, 0)),
        ],
        out_specs=[
            pl.BlockSpec((1, _D, _N), lambda b: (b, 0, 0)),
            pl.BlockSpec((1, 1, _N), lambda b: (b, 0, 0)),
        ],
        out_shape=[
            jax.ShapeDtypeStruct((_B, _D, _N), jnp.float32),
            jax.ShapeDtypeStruct((_B, 1, _N), jnp.int32),
        ],
        compiler_params=pltpu.CompilerParams(
            dimension_semantics=("parallel",)),
    )(x_in, cbm2, e2, cb)
    return out, jnp.reshape(idx, (_B, _N, 1))


# speedup vs baseline: 1.1257x; 1.0640x over previous
import jax
import jax.numpy as jnp
from jax.experimental import pallas as pl
from jax.experimental.pallas import tpu as pltpu

_B, _D, _N = 32, 64, 576
_K = 1024


def _vq_kernel(x_ref, emb_ref, out_ref, idx_ref):
    x = x_ref[0]
    emb = emb_ref[...]
    s = jax.lax.dot_general(
        emb, x, (((1,), (0,)), ((), ())),
        preferred_element_type=jnp.float32)
    e2h = 0.5 * jnp.sum(emb * emb, axis=1, keepdims=True)
    d2 = e2h - s
    idx = jnp.argmin(d2, axis=0)
    idx_ref[0, 0, :] = idx
    onehot = (jax.lax.broadcasted_iota(jnp.int32, (_K, _N), 0)
              == idx[None, :]).astype(jnp.float32)
    out = jax.lax.dot_general(
        emb, onehot, (((0,), (0,)), ((), ())),
        preferred_element_type=jnp.float32)
    out_ref[0] = out


def kernel(x_in, codebook):
    out, idx = pl.pallas_call(
        _vq_kernel,
        grid=(_B,),
        in_specs=[
            pl.BlockSpec((1, _D, _N), lambda b: (b, 0, 0)),
            pl.BlockSpec((_K, _D), lambda b: (0, 0)),
        ],
        out_specs=[
            pl.BlockSpec((1, _D, _N), lambda b: (b, 0, 0)),
            pl.BlockSpec((1, 1, _N), lambda b: (b, 0, 0)),
        ],
        out_shape=[
            jax.ShapeDtypeStruct((_B, _D, _N), jnp.float32),
            jax.ShapeDtypeStruct((_B, 1, _N), jnp.int32),
        ],
        compiler_params=pltpu.CompilerParams(
            dimension_semantics=("parallel",)),
    )(x_in, codebook)
    return out, jnp.reshape(idx, (_B, _N, 1))


# speedup vs baseline: 1.3259x; 1.1778x over previous
import jax
import jax.numpy as jnp
from jax.experimental import pallas as pl
from jax.experimental.pallas import tpu as pltpu

_B, _D, _N = 32, 64, 576
_K = 1024


_BB = 2


def _vq_kernel(x_ref, emb_ref, out_ref, idx_ref):
    emb = emb_ref[...]
    e2h = 0.5 * jnp.sum(emb * emb, axis=1, keepdims=True)
    for b in range(_BB):
        x = x_ref[b]
        s = jax.lax.dot_general(
            emb, x, (((1,), (0,)), ((), ())),
            preferred_element_type=jnp.float32)
        d2 = e2h - s
        idx = jnp.argmin(d2, axis=0)
        idx_ref[b, 0, :] = idx
        onehot = (jax.lax.broadcasted_iota(jnp.int32, (_K, _N), 0)
                  == idx[None, :]).astype(jnp.float32)
        out = jax.lax.dot_general(
            emb, onehot, (((0,), (0,)), ((), ())),
            preferred_element_type=jnp.float32)
        out_ref[b] = out


def kernel(x_in, codebook):
    out, idx = pl.pallas_call(
        _vq_kernel,
        grid=(_B // _BB,),
        in_specs=[
            pl.BlockSpec((_BB, _D, _N), lambda b: (b, 0, 0)),
            pl.BlockSpec((_K, _D), lambda b: (0, 0)),
        ],
        out_specs=[
            pl.BlockSpec((_BB, _D, _N), lambda b: (b, 0, 0)),
            pl.BlockSpec((_BB, 1, _N), lambda b: (b, 0, 0)),
        ],
        out_shape=[
            jax.ShapeDtypeStruct((_B, _D, _N), jnp.float32),
            jax.ShapeDtypeStruct((_B, 1, _N), jnp.int32),
        ],
        compiler_params=pltpu.CompilerParams(
            dimension_semantics=("parallel",)),
    )(x_in, codebook)
    return out, jnp.reshape(idx, (_B, _N, 1))


# speedup vs baseline: 1.4227x; 1.0730x over previous
import jax
import jax.numpy as jnp
from jax.experimental import pallas as pl
from jax.experimental.pallas import tpu as pltpu

_B, _D, _N = 32, 64, 576
_K = 1024


_BB = 4


def _vq_kernel(x_ref, emb_ref, out_ref, idx_ref):
    emb = emb_ref[...]
    e2h = 0.5 * jnp.sum(emb * emb, axis=1, keepdims=True)
    for b in range(_BB):
        x = x_ref[b]
        s = jax.lax.dot_general(
            emb, x, (((1,), (0,)), ((), ())),
            preferred_element_type=jnp.float32)
        d2 = e2h - s
        idx = jnp.argmin(d2, axis=0)
        idx_ref[b, 0, :] = idx
        onehot = (jax.lax.broadcasted_iota(jnp.int32, (_K, _N), 0)
                  == idx[None, :]).astype(jnp.float32)
        out = jax.lax.dot_general(
            emb, onehot, (((0,), (0,)), ((), ())),
            preferred_element_type=jnp.float32)
        out_ref[b] = out


def kernel(x_in, codebook):
    out, idx = pl.pallas_call(
        _vq_kernel,
        grid=(_B // _BB,),
        in_specs=[
            pl.BlockSpec((_BB, _D, _N), lambda b: (b, 0, 0)),
            pl.BlockSpec((_K, _D), lambda b: (0, 0)),
        ],
        out_specs=[
            pl.BlockSpec((_BB, _D, _N), lambda b: (b, 0, 0)),
            pl.BlockSpec((_BB, 1, _N), lambda b: (b, 0, 0)),
        ],
        out_shape=[
            jax.ShapeDtypeStruct((_B, _D, _N), jnp.float32),
            jax.ShapeDtypeStruct((_B, 1, _N), jnp.int32),
        ],
        compiler_params=pltpu.CompilerParams(
            dimension_semantics=("parallel",)),
    )(x_in, codebook)
    return out, jnp.reshape(idx, (_B, _N, 1))


# speedup vs baseline: 1.4718x; 1.0345x over previous
import jax
import jax.numpy as jnp
from jax.experimental import pallas as pl
from jax.experimental.pallas import tpu as pltpu

_B, _D, _N = 32, 64, 576
_K = 1024


_BB = 8


def _vq_kernel(x_ref, emb_ref, out_ref, idx_ref):
    emb = emb_ref[...]
    e2h = 0.5 * jnp.sum(emb * emb, axis=1, keepdims=True)
    for b in range(_BB):
        x = x_ref[b]
        s = jax.lax.dot_general(
            emb, x, (((1,), (0,)), ((), ())),
            preferred_element_type=jnp.float32)
        d2 = e2h - s
        idx = jnp.argmin(d2, axis=0)
        idx_ref[b, 0, :] = idx
        onehot = (jax.lax.broadcasted_iota(jnp.int32, (_K, _N), 0)
                  == idx[None, :]).astype(jnp.float32)
        out = jax.lax.dot_general(
            emb, onehot, (((0,), (0,)), ((), ())),
            preferred_element_type=jnp.float32)
        out_ref[b] = out


def kernel(x_in, codebook):
    out, idx = pl.pallas_call(
        _vq_kernel,
        grid=(_B // _BB,),
        in_specs=[
            pl.BlockSpec((_BB, _D, _N), lambda b: (b, 0, 0)),
            pl.BlockSpec((_K, _D), lambda b: (0, 0)),
        ],
        out_specs=[
            pl.BlockSpec((_BB, _D, _N), lambda b: (b, 0, 0)),
            pl.BlockSpec((_BB, 1, _N), lambda b: (b, 0, 0)),
        ],
        out_shape=[
            jax.ShapeDtypeStruct((_B, _D, _N), jnp.float32),
            jax.ShapeDtypeStruct((_B, 1, _N), jnp.int32),
        ],
        compiler_params=pltpu.CompilerParams(
            dimension_semantics=("parallel",)),
    )(x_in, codebook)
    return out, jnp.reshape(idx, (_B, _N, 1))
